# baseline (speedup 1.0000x reference)
import functools

import jax
import jax.numpy as jnp
import numpy as np
from jax.experimental import pallas as pl
from jax.experimental.pallas import tpu as pltpu

B = 1024
NUM_NODES = 100
EMB = 128
HID = 64
OUT = 128

NSIM = 24


def _band_coefs(n: int) -> np.ndarray:
    deg = np.zeros((n,), dtype=np.float64)
    for i in range(n):
        nbrs = [j for j in (i - 2, i - 1, i + 1, i + 2) if 0 <= j < n]
        deg[i] = len(nbrs) + 1.0
    dis = 1.0 / np.sqrt(deg)
    coef = np.zeros((5, n), dtype=np.float64)
    for s in range(n):
        for d in (-2, -1, 0, 1, 2):
            t = s + d
            if 0 <= t < n:
                coef[2 + d, s] = dis[s] * dis[t]
    return coef.astype(np.float32)


_COEF = _band_coefs(NSIM)
_ROWSUM = _COEF.sum(axis=0)
_A_PACK = np.concatenate([_COEF, _ROWSUM[None]], axis=0)


def _build_edges_const(batch_size: int, num_nodes: int) -> np.ndarray:
    e = []
    for i in range(num_nodes):
        for j in range(i + 1, min(i + 3, num_nodes)):
            e.append((i, j))
            e.append((j, i))
    e = np.asarray(e, dtype=np.int64)
    offs = (np.arange(batch_size, dtype=np.int64) * num_nodes)[:, None, None]
    return (e[None, :, :] + offs).reshape(-1, 2).T


def _expansion_copies(scr_ref, out_ref, sem_ref, step, slot, bb):
    b0 = step * bb
    copies = []

    def add(src_lo, src_hi, dst_lo):
        k = len(copies)
        copies.append(pltpu.make_async_copy(
            scr_ref.at[slot, :, src_lo:src_hi, :],
            out_ref.at[pl.ds(b0, bb), pl.ds(dst_lo, src_hi - src_lo), :],
            sem_ref.at[slot, k]))

    add(0, 16, 0)
    for r in range(9):
        add(8, 16, 16 + 8 * r)
    add(12, 20, 88)
    add(20, 24, 96)
    return copies


def _decoder_block(emb_ref, wt_ref, bt_ref, wc0_ref, bc0_ref, wc1_ref, bc1_ref,
                   wc2_ref, bc2_ref, wo_ref, bo_ref, a_ref, out_ref,
                   scr_ref, sem_ref, *, bb: int):
    band = a_ref[...]
    rowsum = band[5]
    i = pl.program_id(0)
    nsteps = pl.num_programs(0)
    slot = jax.lax.rem(i, 2)

    @pl.when(i >= 2)
    def _():
        for c in _expansion_copies(scr_ref, out_ref, sem_ref, i - 2, slot, bb):
            c.wait()

    def band_apply(t):
        zpad = jnp.zeros((2, bb, HID), dtype=t.dtype)
        tp = jnp.concatenate([zpad, t, zpad], axis=0)
        acc = t * band[2][:, None, None]
        for d in (-2, -1, 1, 2):
            acc = acc + tp[2 + d:2 + d + NSIM] * band[2 + d][:, None, None]
        return acc

    y = jnp.dot(emb_ref[...], wt_ref[...],
                preferred_element_type=jnp.float32) + bt_ref[...]
    t0 = jnp.dot(y, wc0_ref[...], preferred_element_type=jnp.float32)
    x = jax.nn.relu(rowsum[:, None, None] * t0[None, :, :]
                    + bc0_ref[...][None])

    for w_ref, b_ref in ((wc1_ref, bc1_ref), (wc2_ref, bc2_ref)):
        t = jnp.dot(x.reshape(NSIM * bb, HID), w_ref[...],
                    preferred_element_type=jnp.float32)
        t = band_apply(t.reshape(NSIM, bb, HID))
        x = jax.nn.relu(t + b_ref[...][None])

    xb = jnp.swapaxes(x, 0, 1)
    o = jnp.dot(xb.reshape(bb * NSIM, HID), wo_ref[...],
                preferred_element_type=jnp.float32) + bo_ref[...]
    scr_ref[slot] = o.reshape(bb, NSIM, OUT)

    for c in _expansion_copies(scr_ref, out_ref, sem_ref, i, slot, bb):
        c.start()

    @pl.when(i == nsteps - 1)
    def _():
        @pl.when(nsteps >= 2)
        def _():
            for c in _expansion_copies(scr_ref, out_ref, sem_ref,
                                       i - 1, 1 - slot, bb):
                c.wait()
        for c in _expansion_copies(scr_ref, out_ref, sem_ref, i, slot, bb):
            c.wait()


def _decoder(embedding, Wt, bt, Wc0, bc0, Wc1, bc1, Wc2, bc2, Wo, bo, *, bb: int):
    grid = (B // bb,)
    full = lambda i: (0, 0)
    specs = [
        pl.BlockSpec((bb, EMB), lambda i: (i, 0)),
        pl.BlockSpec((EMB, HID), full),
        pl.BlockSpec((1, HID), full),
        pl.BlockSpec((HID, HID), full),
        pl.BlockSpec((1, HID), full),
        pl.BlockSpec((HID, HID), full),
        pl.BlockSpec((1, HID), full),
        pl.BlockSpec((HID, HID), full),
        pl.BlockSpec((1, HID), full),
        pl.BlockSpec((HID, OUT), full),
        pl.BlockSpec((1, OUT), full),
        pl.BlockSpec((6, NSIM), full),
    ]
    return pl.pallas_call(
        functools.partial(_decoder_block, bb=bb),
        grid=grid,
        in_specs=specs,
        out_specs=pl.BlockSpec(memory_space=pltpu.MemorySpace.HBM),
        out_shape=jax.ShapeDtypeStruct((B, NUM_NODES, OUT), jnp.float32),
        scratch_shapes=[
            pltpu.VMEM((2, bb, NSIM, OUT), jnp.float32),
            pltpu.SemaphoreType.DMA((2, 12)),
        ],
        compiler_params=pltpu.CompilerParams(
            dimension_semantics=("arbitrary",)),
    )(embedding, Wt, bt.reshape(1, HID), Wc0, bc0.reshape(1, HID),
      Wc1, bc1.reshape(1, HID), Wc2, bc2.reshape(1, HID), Wo, bo.reshape(1, OUT),
      jnp.asarray(_A_PACK))


_EI_NP = _build_edges_const(B, NUM_NODES)
_BATCH_NP = np.repeat(np.arange(B, dtype=np.int64), NUM_NODES)


def kernel(embedding, target_num_nodes, Wt, bt, Wc0, bc0, Wc1, bc1, Wc2, bc2, Wo, bo):
    out = _decoder(embedding, Wt, bt, Wc0, bc0, Wc1, bc1, Wc2, bc2, Wo, bo, bb=64)
    ei = jnp.asarray(_EI_NP)
    batch_tensor = jnp.asarray(_BATCH_NP)
    batch_tensor = batch_tensor + (jnp.asarray(target_num_nodes)
                                   - NUM_NODES).astype(batch_tensor.dtype)
    return out, ei, batch_tensor


# speedup vs baseline: 1.0589x; 1.0589x over previous
import functools

import jax
import jax.numpy as jnp
import numpy as np
from jax.experimental import pallas as pl
from jax.experimental.pallas import tpu as pltpu

B = 1024
NUM_NODES = 100
EMB = 128
HID = 64
OUT = 128

NSIM = 24


def _band_coefs(n: int) -> np.ndarray:
    deg = np.zeros((n,), dtype=np.float64)
    for i in range(n):
        nbrs = [j for j in (i - 2, i - 1, i + 1, i + 2) if 0 <= j < n]
        deg[i] = len(nbrs) + 1.0
    dis = 1.0 / np.sqrt(deg)
    coef = np.zeros((5, n), dtype=np.float64)
    for s in range(n):
        for d in (-2, -1, 0, 1, 2):
            t = s + d
            if 0 <= t < n:
                coef[2 + d, s] = dis[s] * dis[t]
    return coef.astype(np.float32)


_COEF = _band_coefs(NSIM)
_ROWSUM = _COEF.sum(axis=0)
_A_PACK = np.concatenate([_COEF, _ROWSUM[None]], axis=0)


def _build_edges_const(batch_size: int, num_nodes: int) -> np.ndarray:
    e = []
    for i in range(num_nodes):
        for j in range(i + 1, min(i + 3, num_nodes)):
            e.append((i, j))
            e.append((j, i))
    e = np.asarray(e, dtype=np.int64)
    offs = (np.arange(batch_size, dtype=np.int64) * num_nodes)[:, None, None]
    return (e[None, :, :] + offs).reshape(-1, 2).T


def _make_copy(scr_ref, out_ref, sem_ref, step, slot, bb, k, src_lo, src_hi,
               dst_lo):
    return pltpu.make_async_copy(
        scr_ref.at[slot, :, src_lo:src_hi, :],
        out_ref.at[pl.ds(step * bb, bb), pl.ds(dst_lo, src_hi - src_lo), :],
        sem_ref.at[slot, k])


def _interior_copies(scr_ref, out_ref, sem_ref, step, slot, bb):
    return [_make_copy(scr_ref, out_ref, sem_ref, step, slot, bb, r,
                       8, 16, 8 + 8 * r) for r in range(10)]


def _boundary_copies(scr_ref, out_ref, sem_ref, step, slot, bb):
    mk = lambda k, lo, hi, dst: _make_copy(scr_ref, out_ref, sem_ref, step,
                                           slot, bb, k, lo, hi, dst)
    return [mk(10, 0, 8, 0), mk(11, 12, 20, 88), mk(12, 20, 24, 96)]


def _all_copies(scr_ref, out_ref, sem_ref, step, slot, bb):
    return (_interior_copies(scr_ref, out_ref, sem_ref, step, slot, bb)
            + _boundary_copies(scr_ref, out_ref, sem_ref, step, slot, bb))


def _decoder_block(emb_ref, wt_ref, bt_ref, wc0_ref, bc0_ref, wc1_ref, bc1_ref,
                   wc2_ref, bc2_ref, wo_ref, bo_ref, a_ref, out_ref,
                   scr_ref, sem_ref, *, bb: int):
    band = a_ref[...]
    rowsum = band[5]
    i = pl.program_id(0)
    nsteps = pl.num_programs(0)
    slot = jax.lax.rem(i, 2)

    @pl.when(i >= 2)
    def _():
        for c in _all_copies(scr_ref, out_ref, sem_ref, i - 2, slot, bb):
            c.wait()

    def band_apply(t):
        zpad = jnp.zeros((2, bb, HID), dtype=t.dtype)
        tp = jnp.concatenate([zpad, t, zpad], axis=0)
        acc = t * band[2][:, None, None]
        for d in (-2, -1, 1, 2):
            acc = acc + tp[2 + d:2 + d + NSIM] * band[2 + d][:, None, None]
        return acc

    y = jnp.dot(emb_ref[...], wt_ref[...],
                preferred_element_type=jnp.float32) + bt_ref[...]
    t0 = jnp.dot(y, wc0_ref[...], preferred_element_type=jnp.float32)

    xi = jax.nn.relu(t0 + bc0_ref[...])
    for w_ref, b_ref in ((wc1_ref, bc1_ref), (wc2_ref, bc2_ref)):
        xi = jax.nn.relu(jnp.dot(xi, w_ref[...],
                                 preferred_element_type=jnp.float32)
                         + b_ref[...])
    oi = jnp.dot(xi, wo_ref[...],
                 preferred_element_type=jnp.float32) + bo_ref[...]
    scr_ref[slot, :, 8:16, :] = jnp.broadcast_to(oi[:, None, :], (bb, 8, OUT))
    for c in _interior_copies(scr_ref, out_ref, sem_ref, i, slot, bb):
        c.start()

    x = jax.nn.relu(rowsum[:, None, None] * t0[None, :, :]
                    + bc0_ref[...][None])
    for w_ref, b_ref in ((wc1_ref, bc1_ref), (wc2_ref, bc2_ref)):
        t = jnp.dot(x.reshape(NSIM * bb, HID), w_ref[...],
                    preferred_element_type=jnp.float32)
        t = band_apply(t.reshape(NSIM, bb, HID))
        x = jax.nn.relu(t + b_ref[...][None])

    xb = jnp.swapaxes(x, 0, 1)
    o = jnp.dot(xb.reshape(bb * NSIM, HID), wo_ref[...],
                preferred_element_type=jnp.float32) + bo_ref[...]
    o = o.reshape(bb, NSIM, OUT)
    scr_ref[slot, :, 0:8, :] = o[:, 0:8, :]
    scr_ref[slot, :, 16:24, :] = o[:, 16:24, :]

    for c in _boundary_copies(scr_ref, out_ref, sem_ref, i, slot, bb):
        c.start()

    @pl.when(i == nsteps - 1)
    def _():
        @pl.when(nsteps >= 2)
        def _():
            for c in _all_copies(scr_ref, out_ref, sem_ref,
                                 i - 1, 1 - slot, bb):
                c.wait()
        for c in _all_copies(scr_ref, out_ref, sem_ref, i, slot, bb):
            c.wait()


def _decoder(embedding, Wt, bt, Wc0, bc0, Wc1, bc1, Wc2, bc2, Wo, bo, *, bb: int):
    grid = (B // bb,)
    full = lambda i: (0, 0)
    specs = [
        pl.BlockSpec((bb, EMB), lambda i: (i, 0)),
        pl.BlockSpec((EMB, HID), full),
        pl.BlockSpec((1, HID), full),
        pl.BlockSpec((HID, HID), full),
        pl.BlockSpec((1, HID), full),
        pl.BlockSpec((HID, HID), full),
        pl.BlockSpec((1, HID), full),
        pl.BlockSpec((HID, HID), full),
        pl.BlockSpec((1, HID), full),
        pl.BlockSpec((HID, OUT), full),
        pl.BlockSpec((1, OUT), full),
        pl.BlockSpec((6, NSIM), full),
    ]
    return pl.pallas_call(
        functools.partial(_decoder_block, bb=bb),
        grid=grid,
        in_specs=specs,
        out_specs=pl.BlockSpec(memory_space=pltpu.MemorySpace.HBM),
        out_shape=jax.ShapeDtypeStruct((B, NUM_NODES, OUT), jnp.float32),
        scratch_shapes=[
            pltpu.VMEM((2, bb, NSIM, OUT), jnp.float32),
            pltpu.SemaphoreType.DMA((2, 13)),
        ],
        compiler_params=pltpu.CompilerParams(
            dimension_semantics=("arbitrary",)),
    )(embedding, Wt, bt.reshape(1, HID), Wc0, bc0.reshape(1, HID),
      Wc1, bc1.reshape(1, HID), Wc2, bc2.reshape(1, HID), Wo, bo.reshape(1, OUT),
      jnp.asarray(_A_PACK))


_EI_NP = _build_edges_const(B, NUM_NODES)
_BATCH_NP = np.repeat(np.arange(B, dtype=np.int64), NUM_NODES)


def kernel(embedding, target_num_nodes, Wt, bt, Wc0, bc0, Wc1, bc1, Wc2, bc2, Wo, bo):
    out = _decoder(embedding, Wt, bt, Wc0, bc0, Wc1, bc1, Wc2, bc2, Wo, bo, bb=128)
    ei = jnp.asarray(_EI_NP)
    batch_tensor = jnp.asarray(_BATCH_NP)
    batch_tensor = batch_tensor + (jnp.asarray(target_num_nodes)
                                   - NUM_NODES).astype(batch_tensor.dtype)
    return out, ei, batch_tensor


# speedup vs baseline: 1.0958x; 1.0349x over previous
import functools

import jax
import jax.numpy as jnp
import numpy as np
from jax.experimental import pallas as pl
from jax.experimental.pallas import tpu as pltpu

B = 1024
NUM_NODES = 100
EMB = 128
HID = 64
OUT = 128

NSIM = 24


def _band_coefs(n: int) -> np.ndarray:
    deg = np.zeros((n,), dtype=np.float64)
    for i in range(n):
        nbrs = [j for j in (i - 2, i - 1, i + 1, i + 2) if 0 <= j < n]
        deg[i] = len(nbrs) + 1.0
    dis = 1.0 / np.sqrt(deg)
    coef = np.zeros((5, n), dtype=np.float64)
    for s in range(n):
        for d in (-2, -1, 0, 1, 2):
            t = s + d
            if 0 <= t < n:
                coef[2 + d, s] = dis[s] * dis[t]
    return coef.astype(np.float32)


_COEF = _band_coefs(NSIM)
_ROWSUM = _COEF.sum(axis=0)
_A_PACK = np.concatenate([_COEF, _ROWSUM[None]], axis=0)


def _build_edges_const(batch_size: int, num_nodes: int) -> np.ndarray:
    e = []
    for i in range(num_nodes):
        for j in range(i + 1, min(i + 3, num_nodes)):
            e.append((i, j))
            e.append((j, i))
    e = np.asarray(e, dtype=np.int64)
    offs = (np.arange(batch_size, dtype=np.int64) * num_nodes)[:, None, None]
    return (e[None, :, :] + offs).reshape(-1, 2).T


def _make_copy(scr_ref, out_ref, sem_ref, step, slot, bb, k, src_lo, src_hi,
               dst_lo):
    return pltpu.make_async_copy(
        scr_ref.at[slot, :, src_lo:src_hi, :],
        out_ref.at[pl.ds(step * bb, bb), pl.ds(dst_lo, src_hi - src_lo), :],
        sem_ref.at[slot, k])


def _interior_copies(scr_ref, out_ref, sem_ref, step, slot, bb):
    return [_make_copy(scr_ref, out_ref, sem_ref, step, slot, bb, r,
                       8, 16, 8 + 8 * r) for r in range(10)]


def _boundary_copies(scr_ref, out_ref, sem_ref, step, slot, bb):
    mk = lambda k, lo, hi, dst: _make_copy(scr_ref, out_ref, sem_ref, step,
                                           slot, bb, k, lo, hi, dst)
    return [mk(10, 0, 8, 0), mk(11, 12, 20, 88), mk(12, 20, 24, 96)]


def _all_copies(scr_ref, out_ref, sem_ref, step, slot, bb):
    return (_interior_copies(scr_ref, out_ref, sem_ref, step, slot, bb)
            + _boundary_copies(scr_ref, out_ref, sem_ref, step, slot, bb))


def _decoder_block(emb_ref, wt_ref, bt_ref, wc0_ref, bc0_ref, wc1_ref, bc1_ref,
                   wc2_ref, bc2_ref, wo_ref, bo_ref, a_ref, out_ref,
                   scr_ref, sem_ref, *, bb: int):
    band = a_ref[...]
    rowsum = band[5]
    i = pl.program_id(0)
    nsteps = pl.num_programs(0)
    slot = jax.lax.rem(i, 2)

    @pl.when(i >= 2)
    def _():
        for c in _all_copies(scr_ref, out_ref, sem_ref, i - 2, slot, bb):
            c.wait()

    def band_apply(t):
        zpad = jnp.zeros((2, bb, HID), dtype=t.dtype)
        tp = jnp.concatenate([zpad, t, zpad], axis=0)
        acc = t * band[2][:, None, None]
        for d in (-2, -1, 1, 2):
            acc = acc + tp[2 + d:2 + d + NSIM] * band[2 + d][:, None, None]
        return acc

    y = jnp.dot(emb_ref[...], wt_ref[...],
                preferred_element_type=jnp.float32) + bt_ref[...]
    t0 = jnp.dot(y, wc0_ref[...], preferred_element_type=jnp.float32)

    xi = jax.nn.relu(t0 + bc0_ref[...])
    for w_ref, b_ref in ((wc1_ref, bc1_ref), (wc2_ref, bc2_ref)):
        xi = jax.nn.relu(jnp.dot(xi, w_ref[...],
                                 preferred_element_type=jnp.float32)
                         + b_ref[...])
    oi = jnp.dot(xi, wo_ref[...],
                 preferred_element_type=jnp.float32) + bo_ref[...]
    scr_ref[slot, :, 8:16, :] = jnp.broadcast_to(oi[:, None, :], (bb, 8, OUT))
    for c in _interior_copies(scr_ref, out_ref, sem_ref, i, slot, bb):
        c.start()

    x = jax.nn.relu(rowsum[:, None, None] * t0[None, :, :]
                    + bc0_ref[...][None])
    for w_ref, b_ref in ((wc1_ref, bc1_ref), (wc2_ref, bc2_ref)):
        t = jnp.dot(x.reshape(NSIM * bb, HID), w_ref[...],
                    preferred_element_type=jnp.float32)
        t = band_apply(t.reshape(NSIM, bb, HID))
        x = jax.nn.relu(t + b_ref[...][None])

    xb = jnp.swapaxes(x, 0, 1)
    o = jnp.dot(xb.reshape(bb * NSIM, HID), wo_ref[...],
                preferred_element_type=jnp.float32) + bo_ref[...]
    o = o.reshape(bb, NSIM, OUT)
    scr_ref[slot, :, 0:8, :] = o[:, 0:8, :]
    scr_ref[slot, :, 16:24, :] = o[:, 16:24, :]

    for c in _boundary_copies(scr_ref, out_ref, sem_ref, i, slot, bb):
        c.start()

    @pl.when(i == nsteps - 1)
    def _():
        @pl.when(nsteps >= 2)
        def _():
            for c in _all_copies(scr_ref, out_ref, sem_ref,
                                 i - 1, 1 - slot, bb):
                c.wait()
        for c in _all_copies(scr_ref, out_ref, sem_ref, i, slot, bb):
            c.wait()


def _decoder(embedding, Wt, bt, Wc0, bc0, Wc1, bc1, Wc2, bc2, Wo, bo, *, bb: int):
    grid = (B // bb,)
    full = lambda i: (0, 0)
    specs = [
        pl.BlockSpec((bb, EMB), lambda i: (i, 0)),
        pl.BlockSpec((EMB, HID), full),
        pl.BlockSpec((1, HID), full),
        pl.BlockSpec((HID, HID), full),
        pl.BlockSpec((1, HID), full),
        pl.BlockSpec((HID, HID), full),
        pl.BlockSpec((1, HID), full),
        pl.BlockSpec((HID, HID), full),
        pl.BlockSpec((1, HID), full),
        pl.BlockSpec((HID, OUT), full),
        pl.BlockSpec((1, OUT), full),
        pl.BlockSpec((6, NSIM), full),
    ]
    return pl.pallas_call(
        functools.partial(_decoder_block, bb=bb),
        grid=grid,
        in_specs=specs,
        out_specs=pl.BlockSpec(memory_space=pltpu.MemorySpace.HBM),
        out_shape=jax.ShapeDtypeStruct((B, NUM_NODES, OUT), jnp.float32),
        scratch_shapes=[
            pltpu.VMEM((2, bb, NSIM, OUT), jnp.float32),
            pltpu.SemaphoreType.DMA((2, 13)),
        ],
        compiler_params=pltpu.CompilerParams(
            dimension_semantics=("arbitrary",)),
    )(embedding, Wt, bt.reshape(1, HID), Wc0, bc0.reshape(1, HID),
      Wc1, bc1.reshape(1, HID), Wc2, bc2.reshape(1, HID), Wo, bo.reshape(1, OUT),
      jnp.asarray(_A_PACK))


_EI_NP = _build_edges_const(B, NUM_NODES)
_BATCH_NP = np.repeat(np.arange(B, dtype=np.int64), NUM_NODES)


def kernel(embedding, target_num_nodes, Wt, bt, Wc0, bc0, Wc1, bc1, Wc2, bc2, Wo, bo):
    out = _decoder(embedding, Wt, bt, Wc0, bc0, Wc1, bc1, Wc2, bc2, Wo, bo, bb=256)
    ei = jnp.asarray(_EI_NP)
    batch_tensor = jnp.asarray(_BATCH_NP)
    batch_tensor = batch_tensor + (jnp.asarray(target_num_nodes)
                                   - NUM_NODES).astype(batch_tensor.dtype)
    return out, ei, batch_tensor


# speedup vs baseline: 1.0986x; 1.0025x over previous
import functools

import jax
import jax.numpy as jnp
import numpy as np
from jax.experimental import pallas as pl
from jax.experimental.pallas import tpu as pltpu

B = 1024
NUM_NODES = 100
EMB = 128
HID = 64
OUT = 128

NSIM = 24


def _band_coefs(n: int) -> np.ndarray:
    deg = np.zeros((n,), dtype=np.float64)
    for i in range(n):
        nbrs = [j for j in (i - 2, i - 1, i + 1, i + 2) if 0 <= j < n]
        deg[i] = len(nbrs) + 1.0
    dis = 1.0 / np.sqrt(deg)
    coef = np.zeros((5, n), dtype=np.float64)
    for s in range(n):
        for d in (-2, -1, 0, 1, 2):
            t = s + d
            if 0 <= t < n:
                coef[2 + d, s] = dis[s] * dis[t]
    return coef.astype(np.float32)


_COEF = _band_coefs(NSIM)
_ROWSUM = _COEF.sum(axis=0)
_A_PACK = np.concatenate([_COEF, _ROWSUM[None]], axis=0)


def _build_edges_const(batch_size: int, num_nodes: int) -> np.ndarray:
    e = []
    for i in range(num_nodes):
        for j in range(i + 1, min(i + 3, num_nodes)):
            e.append((i, j))
            e.append((j, i))
    e = np.asarray(e, dtype=np.int64)
    offs = (np.arange(batch_size, dtype=np.int64) * num_nodes)[:, None, None]
    return (e[None, :, :] + offs).reshape(-1, 2).T


def _make_copy(scr_ref, out_ref, sem_ref, step, slot, bb, k, src_lo, src_hi,
               dst_lo):
    return pltpu.make_async_copy(
        scr_ref.at[slot, :, src_lo:src_hi, :],
        out_ref.at[pl.ds(step * bb, bb), pl.ds(dst_lo, src_hi - src_lo), :],
        sem_ref.at[slot, k])


def _interior_copies(scr_ref, out_ref, sem_ref, step, slot, bb):
    return [_make_copy(scr_ref, out_ref, sem_ref, step, slot, bb, r,
                       8, 16, 8 + 8 * r) for r in range(10)]


def _boundary_copies(scr_ref, out_ref, sem_ref, step, slot, bb):
    mk = lambda k, lo, hi, dst: _make_copy(scr_ref, out_ref, sem_ref, step,
                                           slot, bb, k, lo, hi, dst)
    return [mk(10, 0, 8, 0), mk(11, 12, 20, 88), mk(12, 20, 24, 96)]


def _all_copies(scr_ref, out_ref, sem_ref, step, slot, bb):
    return (_interior_copies(scr_ref, out_ref, sem_ref, step, slot, bb)
            + _boundary_copies(scr_ref, out_ref, sem_ref, step, slot, bb))


def _decoder_block(emb_ref, wt_ref, bt_ref, wc0_ref, bc0_ref, wc1_ref, bc1_ref,
                   wc2_ref, bc2_ref, wo_ref, bo_ref, a_ref, out_ref,
                   scr_ref, sem_ref, *, bb: int):
    band = a_ref[...]
    rowsum = band[5]
    i = pl.program_id(0)
    nsteps = pl.num_programs(0)
    slot = jax.lax.rem(i, 2)

    @pl.when(i >= 2)
    def _():
        for c in _all_copies(scr_ref, out_ref, sem_ref, i - 2, slot, bb):
            c.wait()

    def band_apply(t):
        zpad = jnp.zeros((2, bb, HID), dtype=t.dtype)
        tp = jnp.concatenate([zpad, t, zpad], axis=0)
        acc = t * band[2][:, None, None]
        for d in (-2, -1, 1, 2):
            acc = acc + tp[2 + d:2 + d + NSIM] * band[2 + d][:, None, None]
        return acc

    y = jnp.dot(emb_ref[...], wt_ref[...],
                preferred_element_type=jnp.float32) + bt_ref[...]
    t0 = jnp.dot(y, wc0_ref[...], preferred_element_type=jnp.float32)

    xi = jax.nn.relu(t0 + bc0_ref[...])
    for w_ref, b_ref in ((wc1_ref, bc1_ref), (wc2_ref, bc2_ref)):
        xi = jax.nn.relu(jnp.dot(xi, w_ref[...],
                                 preferred_element_type=jnp.float32)
                         + b_ref[...])
    oi = jnp.dot(xi, wo_ref[...],
                 preferred_element_type=jnp.float32) + bo_ref[...]
    scr_ref[slot, :, 8:16, :] = jnp.broadcast_to(oi[:, None, :], (bb, 8, OUT))
    for c in _interior_copies(scr_ref, out_ref, sem_ref, i, slot, bb):
        c.start()

    x = jax.nn.relu(rowsum[:, None, None] * t0[None, :, :]
                    + bc0_ref[...][None])
    for w_ref, b_ref in ((wc1_ref, bc1_ref), (wc2_ref, bc2_ref)):
        t = jnp.dot(x.reshape(NSIM * bb, HID), w_ref[...],
                    preferred_element_type=jnp.float32)
        t = band_apply(t.reshape(NSIM, bb, HID))
        x = jax.nn.relu(t + b_ref[...][None])

    xb = jnp.swapaxes(x, 0, 1)
    o = jnp.dot(xb.reshape(bb * NSIM, HID), wo_ref[...],
                preferred_element_type=jnp.float32) + bo_ref[...]
    o = o.reshape(bb, NSIM, OUT)
    scr_ref[slot, :, 0:8, :] = o[:, 0:8, :]
    scr_ref[slot, :, 16:24, :] = o[:, 16:24, :]

    for c in _boundary_copies(scr_ref, out_ref, sem_ref, i, slot, bb):
        c.start()

    @pl.when(i == nsteps - 1)
    def _():
        @pl.when(nsteps >= 2)
        def _():
            for c in _all_copies(scr_ref, out_ref, sem_ref,
                                 i - 1, 1 - slot, bb):
                c.wait()
        for c in _all_copies(scr_ref, out_ref, sem_ref, i, slot, bb):
            c.wait()


def _decoder(embedding, Wt, bt, Wc0, bc0, Wc1, bc1, Wc2, bc2, Wo, bo, *, bb: int):
    grid = (B // bb,)
    full = lambda i: (0, 0)
    specs = [
        pl.BlockSpec((bb, EMB), lambda i: (i, 0)),
        pl.BlockSpec((EMB, HID), full),
        pl.BlockSpec((1, HID), full),
        pl.BlockSpec((HID, HID), full),
        pl.BlockSpec((1, HID), full),
        pl.BlockSpec((HID, HID), full),
        pl.BlockSpec((1, HID), full),
        pl.BlockSpec((HID, HID), full),
        pl.BlockSpec((1, HID), full),
        pl.BlockSpec((HID, OUT), full),
        pl.BlockSpec((1, OUT), full),
        pl.BlockSpec((6, NSIM), full),
    ]
    return pl.pallas_call(
        functools.partial(_decoder_block, bb=bb),
        grid=grid,
        in_specs=specs,
        out_specs=pl.BlockSpec(memory_space=pltpu.MemorySpace.HBM),
        out_shape=jax.ShapeDtypeStruct((B, NUM_NODES, OUT), jnp.float32),
        scratch_shapes=[
            pltpu.VMEM((2, bb, NSIM, OUT), jnp.float32),
            pltpu.SemaphoreType.DMA((2, 13)),
        ],
        compiler_params=pltpu.CompilerParams(
            dimension_semantics=("arbitrary",)),
    )(embedding, Wt, bt.reshape(1, HID), Wc0, bc0.reshape(1, HID),
      Wc1, bc1.reshape(1, HID), Wc2, bc2.reshape(1, HID), Wo, bo.reshape(1, OUT),
      jnp.asarray(_A_PACK))


_EI_NP = _build_edges_const(B, NUM_NODES)
_BATCH_NP = np.repeat(np.arange(B, dtype=np.int64), NUM_NODES)


def kernel(embedding, target_num_nodes, Wt, bt, Wc0, bc0, Wc1, bc1, Wc2, bc2, Wo, bo):
    out = _decoder(embedding, Wt, bt, Wc0, bc0, Wc1, bc1, Wc2, bc2, Wo, bo, bb=512)
    ei = jnp.asarray(_EI_NP)
    batch_tensor = jnp.asarray(_BATCH_NP)
    batch_tensor = batch_tensor + (jnp.asarray(target_num_nodes)
                                   - NUM_NODES).astype(batch_tensor.dtype)
    return out, ei, batch_tensor
